# baseline (speedup 1.0000x reference)
import jax
import jax.numpy as jnp
from jax.experimental import pallas as pl
from jax.experimental.pallas import tpu as pltpu

_TB = 256


def _body(gid_ref, x_ref, w1_hbm, w2_hbm, b2_hbm, exp_hbm, o_ref,
          w1_v, w2_v, b2_v, exp_v, sem):
    i = pl.program_id(0)

    @pl.when(i == 0)
    def _load_weights():
        for src, dst in ((w1_hbm, w1_v), (w2_hbm, w2_v),
                         (b2_hbm, b2_v), (exp_hbm, exp_v)):
            cp = pltpu.make_async_copy(src, dst, sem)
            cp.start()
            cp.wait()

    E = b2_v.shape[0]
    TB, n_agents, d = x_ref.shape
    B = TB * n_agents
    A = o_ref.shape[2]
    eids = jax.lax.broadcasted_iota(jnp.int32, (1, E), 1)
    xall = x_ref[...].reshape(B, d)
    g = gid_ref[...]
    onehot = jnp.where(g == eids, 1.0, 0.0).astype(jnp.bfloat16)
    SUB = 512
    TSUB = SUB // n_agents
    for s in range(B // SUB):
        sl = slice(s * SUB, (s + 1) * SUB)
        x = xall[sl, :].astype(jnp.bfloat16)
        oh = onehot[sl, :]
        hpre = jnp.dot(x, w1_v[...], preferred_element_type=jnp.float32)
        bm = jnp.dot(oh, exp_v[...], preferred_element_type=jnp.float32)
        h = jnp.maximum(hpre + bm, 0.0)
        b2sel = jnp.dot(oh, b2_v[...],
                        preferred_element_type=jnp.float32)
        hb = h.astype(jnp.bfloat16)
        out = jnp.dot(hb, w2_v[...], preferred_element_type=jnp.float32)
        o_ref[s * TSUB:(s + 1) * TSUB, :, :] = (
            (out + b2sel).reshape(TSUB, n_agents, A))


def kernel(obs, group_ids, W1, b1, W2, b2, ph_to_feature):
    n_threads, n_agents, d = obs.shape
    E, dp1, H = W1.shape
    A = W2.shape[2]
    nb = n_threads // _TB
    EH = E * H

    b1eff = (b1 + ph_to_feature * W1[:, d, :]).reshape(1, EH)
    W1all = jnp.transpose(W1[:, :d, :], (1, 0, 2)).reshape(d, EH)
    W1all = W1all.astype(jnp.bfloat16)
    W2all = W2.reshape(EH, A).astype(jnp.bfloat16)
    b2b = b2.astype(jnp.bfloat16)
    col_e = (jnp.arange(EH, dtype=jnp.int32) // H)[None, :]
    row_e = jnp.arange(E, dtype=jnp.int32)[:, None]
    biasmask = jnp.where(row_e == col_e, b1eff, -1e30).astype(jnp.bfloat16)

    any_spec = pl.BlockSpec(memory_space=pl.ANY)
    out = pl.pallas_call(
        _body,
        grid=(nb,),
        in_specs=[
            pl.BlockSpec((_TB * n_agents, 1), lambda i: (i, 0)),
            pl.BlockSpec((_TB, n_agents, d), lambda i: (i, 0, 0)),
            any_spec, any_spec, any_spec, any_spec,
        ],
        out_specs=pl.BlockSpec((_TB, n_agents, A), lambda i: (i, 0, 0)),
        out_shape=jax.ShapeDtypeStruct((n_threads, n_agents, A), jnp.float32),
        scratch_shapes=[
            pltpu.VMEM((d, EH), jnp.bfloat16),
            pltpu.VMEM((EH, A), jnp.bfloat16),
            pltpu.VMEM((E, A), jnp.bfloat16),
            pltpu.VMEM((E, EH), jnp.bfloat16),
            pltpu.SemaphoreType.DMA,
        ],
        compiler_params=pltpu.CompilerParams(
            dimension_semantics=("arbitrary",),
        ),
    )(group_ids.reshape(n_threads * n_agents, 1), obs,
      W1all, W2all, b2b, biasmask)
    return out


# speedup vs baseline: 1.1164x; 1.1164x over previous
import jax
import jax.numpy as jnp
from jax.experimental import pallas as pl
from jax.experimental.pallas import tpu as pltpu

_TB = 256


def _body(gid_ref, x_ref, w1_hbm, w2_hbm, b2_hbm, exp_hbm, o_ref,
          w1_v, w2_v, b2_v, exp_v, sem):
    i = pl.program_id(0)

    @pl.when(i == 0)
    def _load_weights():
        for src, dst in ((w1_hbm, w1_v), (w2_hbm, w2_v),
                         (b2_hbm, b2_v), (exp_hbm, exp_v)):
            cp = pltpu.make_async_copy(src, dst, sem)
            cp.start()
            cp.wait()

    E = b2_v.shape[0]
    TB, n_agents, d = x_ref.shape
    B = TB * n_agents
    A = o_ref.shape[2]
    eids = jax.lax.broadcasted_iota(jnp.int32, (1, 1, E), 2)
    xall = x_ref[...].reshape(B, d)
    g3 = gid_ref[...][:, :, None]
    onehot = jnp.where(g3 == eids, 1.0, 0.0).astype(jnp.bfloat16)
    onehot = onehot.reshape(B, E)
    SUB = 512
    TSUB = SUB // n_agents
    for s in range(B // SUB):
        sl = slice(s * SUB, (s + 1) * SUB)
        x = xall[sl, :].astype(jnp.bfloat16)
        oh = onehot[sl, :]
        hpre = jnp.dot(x, w1_v[...], preferred_element_type=jnp.float32)
        bm = jnp.dot(oh, exp_v[...], preferred_element_type=jnp.float32)
        h = jnp.maximum(hpre + bm, 0.0)
        b2sel = jnp.dot(oh, b2_v[...],
                        preferred_element_type=jnp.float32)
        hb = h.astype(jnp.bfloat16)
        out = jnp.dot(hb, w2_v[...], preferred_element_type=jnp.float32)
        o_ref[s * TSUB:(s + 1) * TSUB, :, :] = (
            (out + b2sel).reshape(TSUB, n_agents, A))


def kernel(obs, group_ids, W1, b1, W2, b2, ph_to_feature):
    n_threads, n_agents, d = obs.shape
    E, dp1, H = W1.shape
    A = W2.shape[2]
    nb = n_threads // _TB
    EH = E * H

    b1eff = (b1 + ph_to_feature * W1[:, d, :]).reshape(1, EH)
    W1all = jnp.transpose(W1[:, :d, :], (1, 0, 2)).reshape(d, EH)
    W1all = W1all.astype(jnp.bfloat16)
    W2all = W2.reshape(EH, A).astype(jnp.bfloat16)
    b2b = b2.astype(jnp.bfloat16)
    col_e = (jnp.arange(EH, dtype=jnp.int32) // H)[None, :]
    row_e = jnp.arange(E, dtype=jnp.int32)[:, None]
    biasmask = jnp.where(row_e == col_e, b1eff, -1e30).astype(jnp.bfloat16)

    any_spec = pl.BlockSpec(memory_space=pl.ANY)
    out = pl.pallas_call(
        _body,
        grid=(nb,),
        in_specs=[
            pl.BlockSpec((_TB, n_agents), lambda i: (i, 0)),
            pl.BlockSpec((_TB, n_agents, d), lambda i: (i, 0, 0)),
            any_spec, any_spec, any_spec, any_spec,
        ],
        out_specs=pl.BlockSpec((_TB, n_agents, A), lambda i: (i, 0, 0)),
        out_shape=jax.ShapeDtypeStruct((n_threads, n_agents, A), jnp.float32),
        scratch_shapes=[
            pltpu.VMEM((d, EH), jnp.bfloat16),
            pltpu.VMEM((EH, A), jnp.bfloat16),
            pltpu.VMEM((E, A), jnp.bfloat16),
            pltpu.VMEM((E, EH), jnp.bfloat16),
            pltpu.SemaphoreType.DMA,
        ],
        compiler_params=pltpu.CompilerParams(
            dimension_semantics=("arbitrary",),
        ),
    )(group_ids, obs, W1all, W2all, b2b, biasmask)
    return out


# speedup vs baseline: 1.1393x; 1.0205x over previous
import jax
import jax.numpy as jnp
from jax.experimental import pallas as pl
from jax.experimental.pallas import tpu as pltpu

_TB = 512


def _body(gid_ref, x_ref, w1_hbm, w2_hbm, b2_hbm, exp_hbm, o_ref,
          w1_v, w2_v, b2_v, exp_v, sem):
    i = pl.program_id(0)

    @pl.when(i == 0)
    def _load_weights():
        for src, dst in ((w1_hbm, w1_v), (w2_hbm, w2_v),
                         (b2_hbm, b2_v), (exp_hbm, exp_v)):
            cp = pltpu.make_async_copy(src, dst, sem)
            cp.start()
            cp.wait()

    E = b2_v.shape[0]
    TB, n_agents, d = x_ref.shape
    B = TB * n_agents
    A = o_ref.shape[2]
    eids = jax.lax.broadcasted_iota(jnp.int32, (1, 1, E), 2)
    xall = x_ref[...].reshape(B, d)
    g3 = gid_ref[...][:, :, None]
    onehot = jnp.where(g3 == eids, 1.0, 0.0).astype(jnp.bfloat16)
    onehot = onehot.reshape(B, E)
    SUB = 512
    TSUB = SUB // n_agents
    for s in range(B // SUB):
        sl = slice(s * SUB, (s + 1) * SUB)
        x = xall[sl, :].astype(jnp.bfloat16)
        oh = onehot[sl, :]
        hpre = jnp.dot(x, w1_v[...], preferred_element_type=jnp.float32)
        bm = jnp.dot(oh, exp_v[...], preferred_element_type=jnp.float32)
        h = jnp.maximum(hpre + bm, 0.0)
        b2sel = jnp.dot(oh, b2_v[...],
                        preferred_element_type=jnp.float32)
        hb = h.astype(jnp.bfloat16)
        out = jnp.dot(hb, w2_v[...], preferred_element_type=jnp.float32)
        o_ref[s * TSUB:(s + 1) * TSUB, :, :] = (
            (out + b2sel).reshape(TSUB, n_agents, A))


def kernel(obs, group_ids, W1, b1, W2, b2, ph_to_feature):
    n_threads, n_agents, d = obs.shape
    E, dp1, H = W1.shape
    A = W2.shape[2]
    nb = n_threads // _TB
    EH = E * H

    b1eff = (b1 + ph_to_feature * W1[:, d, :]).reshape(1, EH)
    W1all = jnp.transpose(W1[:, :d, :], (1, 0, 2)).reshape(d, EH)
    W1all = W1all.astype(jnp.bfloat16)
    W2all = W2.reshape(EH, A).astype(jnp.bfloat16)
    b2b = b2.astype(jnp.bfloat16)
    col_e = (jnp.arange(EH, dtype=jnp.int32) // H)[None, :]
    row_e = jnp.arange(E, dtype=jnp.int32)[:, None]
    biasmask = jnp.where(row_e == col_e, b1eff, -1e30).astype(jnp.bfloat16)

    any_spec = pl.BlockSpec(memory_space=pl.ANY)
    out = pl.pallas_call(
        _body,
        grid=(nb,),
        in_specs=[
            pl.BlockSpec((_TB, n_agents), lambda i: (i, 0)),
            pl.BlockSpec((_TB, n_agents, d), lambda i: (i, 0, 0)),
            any_spec, any_spec, any_spec, any_spec,
        ],
        out_specs=pl.BlockSpec((_TB, n_agents, A), lambda i: (i, 0, 0)),
        out_shape=jax.ShapeDtypeStruct((n_threads, n_agents, A), jnp.float32),
        scratch_shapes=[
            pltpu.VMEM((d, EH), jnp.bfloat16),
            pltpu.VMEM((EH, A), jnp.bfloat16),
            pltpu.VMEM((E, A), jnp.bfloat16),
            pltpu.VMEM((E, EH), jnp.bfloat16),
            pltpu.SemaphoreType.DMA,
        ],
        compiler_params=pltpu.CompilerParams(
            dimension_semantics=("arbitrary",),
        ),
    )(group_ids, obs, W1all, W2all, b2b, biasmask)
    return out
